# baseline (speedup 1.0000x reference)
import jax
import jax.numpy as jnp
from jax.experimental import pallas as pl
from jax.experimental.pallas import tpu as pltpu

ALPHA = 0.2
BLOCK = 16384


def _fused_body(x_ref, cache_ref, a_ref, b_ref, o_ref):
    x = x_ref[...]
    x16 = x.astype(jnp.bfloat16)
    cache = cache_ref[...]
    raw = jax.lax.dot_general(
        x16, cache, (((1,), (1,)), ((), ())), preferred_element_type=jnp.float32
    )
    ssq = jnp.sum(x * x, axis=1, keepdims=True)
    s = raw * jax.lax.rsqrt(jnp.maximum(jnp.broadcast_to(ssq, raw.shape), 1e-24))
    e = jnp.exp(s)
    p = (e * (1.0 / jnp.sum(e, axis=1, keepdims=True))).astype(jnp.bfloat16)
    fine = jnp.dot(p, cache, preferred_element_type=jnp.float32)
    o_ref[...] = (
        jnp.dot(x16, a_ref[...], preferred_element_type=jnp.float32)
        + jnp.dot(fine.astype(jnp.bfloat16), b_ref[...], preferred_element_type=jnp.float32)
    )


def kernel(text_token, cache, W):
    n_rows, d = text_token.shape
    m = cache.shape[0]
    a = (ALPHA * W[:, :d].T + jnp.eye(d, dtype=W.dtype)).astype(jnp.bfloat16)
    b = (ALPHA * W[:, d:].T).astype(jnp.bfloat16)
    cache = cache.astype(jnp.bfloat16)
    out = pl.pallas_call(
        _fused_body,
        grid=(n_rows // BLOCK,),
        in_specs=[
            pl.BlockSpec((BLOCK, d), lambda i: (i, 0)),
            pl.BlockSpec((m, d), lambda i: (0, 0)),
            pl.BlockSpec((d, d), lambda i: (0, 0)),
            pl.BlockSpec((d, d), lambda i: (0, 0)),
        ],
        out_specs=pl.BlockSpec((BLOCK, d), lambda i: (i, 0)),
        out_shape=jax.ShapeDtypeStruct((n_rows, d), text_token.dtype),
        compiler_params=pltpu.CompilerParams(
            dimension_semantics=("parallel",),
        ),
    )(text_token, cache, a, b)
    return (out, 0.0)


# speedup vs baseline: 1.0993x; 1.0993x over previous
import jax
import jax.numpy as jnp
from jax.experimental import pallas as pl
from jax.experimental.pallas import tpu as pltpu

ALPHA = 0.2
BLOCK = 16384


def _fused_body(x_ref, cache_ref, a_ref, b_ref, o_ref):
    x = x_ref[...]
    x16 = x.astype(jnp.bfloat16)
    cache = cache_ref[...]
    raw = jax.lax.dot_general(
        x16, cache, (((1,), (1,)), ((), ())), preferred_element_type=jnp.float32
    )
    ssq = jnp.sum(x * x, axis=1, keepdims=True)
    s = raw * jax.lax.rsqrt(jnp.maximum(jnp.broadcast_to(ssq, raw.shape), 1e-24))
    e = jnp.exp(s)
    p = (e * (1.0 / jnp.sum(e, axis=1, keepdims=True))).astype(jnp.bfloat16)
    o_ref[...] = (
        jnp.dot(x16, a_ref[...], preferred_element_type=jnp.float32)
        + jnp.dot(p, b_ref[...], preferred_element_type=jnp.float32)
    )


def kernel(text_token, cache, W):
    n_rows, d = text_token.shape
    m = cache.shape[0]
    a = (ALPHA * W[:, :d].T + jnp.eye(d, dtype=W.dtype)).astype(jnp.bfloat16)
    b = (cache @ (ALPHA * W[:, d:].T)).astype(jnp.bfloat16)
    cache = cache.astype(jnp.bfloat16)
    out = pl.pallas_call(
        _fused_body,
        grid=(n_rows // BLOCK,),
        in_specs=[
            pl.BlockSpec((BLOCK, d), lambda i: (i, 0)),
            pl.BlockSpec((m, d), lambda i: (0, 0)),
            pl.BlockSpec((d, d), lambda i: (0, 0)),
            pl.BlockSpec((m, d), lambda i: (0, 0)),
        ],
        out_specs=pl.BlockSpec((BLOCK, d), lambda i: (i, 0)),
        out_shape=jax.ShapeDtypeStruct((n_rows, d), text_token.dtype),
        compiler_params=pltpu.CompilerParams(
            dimension_semantics=("parallel",),
        ),
    )(text_token, cache, a, b)
    return (out, 0.0)
